# baseline (speedup 1.0000x reference)
import functools

import numpy as np

import jax
import jax.numpy as jnp
from jax import lax
from jax.experimental import pallas as pl
from jax.experimental.pallas import tpu as pltpu
from jax.experimental.pallas import tpu_sc as plsc

VOCAB = 32000
D_MODEL = 512
BATCH = 64
SEQ = 512

NC = 2
NS = 16
NW = NC * NS
ROWS = BATCH * SEQ
RPW = ROWS // NW
CHUNK = 32
NCHUNK = RPW // CHUNK
PCHUNK = SEQ // CHUNK
LANES = 16
GRP = D_MODEL // LANES


def _positional_encoding():
    pos = np.arange(SEQ, dtype=np.float32)[:, None]
    i = np.arange(D_MODEL, dtype=np.float32)[None, :]
    angle = (pos / np.power(np.float32(10000.0),
                            np.float32(2.0) * i / np.float32(D_MODEL),
                            dtype=np.float32)).astype(np.float32)
    even = (np.arange(D_MODEL) % 2 == 0)[None, :]
    return np.where(even, np.sin(angle), np.cos(angle)).astype(np.float32)


_mesh = plsc.VectorSubcoreMesh(core_axis_name="c", subcore_axis_name="s")


def _chunk_of(i):
    return (i % 2) * PCHUNK + i // 2


def _pe_off(i):
    return (i // 2) * CHUNK


@functools.partial(
    pl.kernel,
    mesh=_mesh,
    out_type=jax.ShapeDtypeStruct((ROWS, D_MODEL), jnp.float32),
    scratch_types=[
        pltpu.VMEM((2, SEQ), jnp.int32),
        pltpu.VMEM((CHUNK, D_MODEL), jnp.float32),
        pltpu.VMEM((CHUNK, D_MODEL), jnp.float32),
        pltpu.VMEM((CHUNK, D_MODEL), jnp.float32),
        pltpu.VMEM((CHUNK, D_MODEL), jnp.float32),
        pltpu.VMEM((CHUNK, D_MODEL), jnp.float32),
        pltpu.VMEM((CHUNK, D_MODEL), jnp.float32),
        pltpu.VMEM_SHARED((SEQ, D_MODEL), jnp.float32),
        pltpu.SemaphoreType.DMA,
        pltpu.SemaphoreType.DMA,
        pltpu.SemaphoreType.DMA,
        pltpu.SemaphoreType.DMA,
        pltpu.SemaphoreType.DMA,
        pltpu.SemaphoreType.DMA,
    ],
)
def _emb_kernel(x_hbm, table_hbm, pe_hbm, out_hbm, idx_v,
                g0, g1, o0, o1, o2, o3, pe_sh,
                gs0, gs1, os0, os1, os2, os3):
    wid = lax.axis_index("s") * NC + lax.axis_index("c")
    sid = lax.axis_index("s")
    base = wid * RPW
    g = (g0, g1)
    o = (o0, o1, o2, o3)
    gsem = (gs0, gs1)
    osem = (os0, os1, os2, os3)

    pltpu.sync_copy(x_hbm.at[pl.ds(2 * wid, 2)], idx_v)

    @pl.when(sid == 0)
    def _():
        pltpu.sync_copy(pe_hbm, pe_sh)

    plsc.subcore_barrier()

    hg = {}
    ho = {}
    hpf = {}
    for i in range(4):
        hpf[i] = pltpu.async_copy(
            pe_sh.at[pl.ds(_pe_off(i), CHUNK)], o[i % 4], osem[i % 4])
    for i in range(2):
        hg[i] = pltpu.async_copy(
            table_hbm.at[idx_v.at[i % 2, pl.ds((i // 2) * CHUNK, CHUNK)]],
            g[i], gsem[i])

    for i in range(NCHUNK):
        b2 = i % 2
        b4 = i % 4
        hg[i].wait()
        hpf[i].wait()

        def addbody(r, carry, _b2=b2, _b4=b4):
            for jg in range(GRP):
                sl = pl.ds(jg * LANES, LANES)
                plsc.addupdate(o[_b4].at[r, sl], g[_b2][r, sl])
            return carry

        lax.fori_loop(0, CHUNK, addbody, 0)

        if i + 2 < NCHUNK:
            hg[i + 2] = pltpu.async_copy(
                table_hbm.at[
                    idx_v.at[(i + 2) % 2, pl.ds(((i + 2) // 2) * CHUNK, CHUNK)]],
                g[b2], gsem[b2])
        ho[i] = pltpu.async_copy(
            o[b4], out_hbm.at[pl.ds(base + _chunk_of(i) * CHUNK, CHUNK)],
            osem[b4])
        if 2 <= i and i + 2 < NCHUNK:
            ho[i - 2].wait()
            hpf[i + 2] = pltpu.async_copy(
                pe_sh.at[pl.ds(_pe_off(i + 2), CHUNK)],
                o[(i + 2) % 4], osem[(i + 2) % 4])

    ho[NCHUNK - 4].wait()
    ho[NCHUNK - 3].wait()
    ho[NCHUNK - 2].wait()
    ho[NCHUNK - 1].wait()


def kernel(x, table):
    pe = _positional_encoding()
    out = _emb_kernel(x.astype(jnp.int32), table, pe)
    return out.reshape(BATCH, SEQ, D_MODEL)


# speedup vs baseline: 1.4826x; 1.4826x over previous
import functools

import numpy as np

import jax
import jax.numpy as jnp
from jax import lax
from jax.experimental import pallas as pl
from jax.experimental.pallas import tpu as pltpu
from jax.experimental.pallas import tpu_sc as plsc

VOCAB = 32000
D_MODEL = 512
BATCH = 64
SEQ = 512

NC = 2
NS = 16
NW = NC * NS
ROWS = BATCH * SEQ
RPW = ROWS // NW
CHUNK = 32
NCHUNK = RPW // CHUNK
PCHUNK = SEQ // CHUNK
LANES = 16
GRP = D_MODEL // LANES


def _positional_encoding():
    pos = np.arange(SEQ, dtype=np.float32)[:, None]
    i = np.arange(D_MODEL, dtype=np.float32)[None, :]
    angle = (pos / np.power(np.float32(10000.0),
                            np.float32(2.0) * i / np.float32(D_MODEL),
                            dtype=np.float32)).astype(np.float32)
    even = (np.arange(D_MODEL) % 2 == 0)[None, :]
    return np.where(even, np.sin(angle), np.cos(angle)).astype(np.float32)


_mesh = plsc.VectorSubcoreMesh(core_axis_name="c", subcore_axis_name="s")


def _chunk_of(i):
    return (i % 2) * PCHUNK + i // 2


def _pe_off(i):
    return (i // 2) * CHUNK


@functools.partial(
    pl.kernel,
    mesh=_mesh,
    out_type=jax.ShapeDtypeStruct((ROWS, D_MODEL), jnp.float32),
    scratch_types=[
        pltpu.VMEM((NCHUNK, CHUNK), jnp.int32),
        pltpu.VMEM((CHUNK, D_MODEL), jnp.float32),
        pltpu.VMEM((CHUNK, D_MODEL), jnp.float32),
        pltpu.VMEM((CHUNK, D_MODEL), jnp.float32),
        pltpu.VMEM((CHUNK, D_MODEL), jnp.float32),
        pltpu.VMEM((CHUNK, D_MODEL), jnp.float32),
        pltpu.VMEM((CHUNK, D_MODEL), jnp.float32),
        pltpu.VMEM_SHARED((SEQ, D_MODEL), jnp.float32),
        pltpu.SemaphoreType.DMA,
        pltpu.SemaphoreType.DMA,
        pltpu.SemaphoreType.DMA,
        pltpu.SemaphoreType.DMA,
        pltpu.SemaphoreType.DMA,
        pltpu.SemaphoreType.DMA,
    ],
)
def _emb_kernel(x_hbm, table_hbm, pe_hbm, out_hbm, idx_v,
                g0, g1, o0, o1, o2, o3, pe_sh,
                gs0, gs1, os0, os1, os2, os3):
    wid = lax.axis_index("s") * NC + lax.axis_index("c")
    sid = lax.axis_index("s")
    base = wid * RPW
    g = (g0, g1)
    o = (o0, o1, o2, o3)
    gsem = (gs0, gs1)
    osem = (os0, os1, os2, os3)

    pltpu.sync_copy(x_hbm.at[wid], idx_v)

    @pl.when(sid == 0)
    def _():
        pltpu.sync_copy(pe_hbm, pe_sh)

    plsc.subcore_barrier()

    hg = {}
    ho = {}
    hpf = {}
    for i in range(4):
        hpf[i] = pltpu.async_copy(
            pe_sh.at[pl.ds(_pe_off(i), CHUNK)], o[i % 4], osem[i % 4])
    for i in range(2):
        hg[i] = pltpu.async_copy(
            table_hbm.at[idx_v.at[_chunk_of(i)]], g[i], gsem[i])

    for i in range(NCHUNK):
        b2 = i % 2
        b4 = i % 4
        hg[i].wait()
        hpf[i].wait()

        def addbody(r, carry, _b2=b2, _b4=b4):
            for jg in range(GRP):
                sl = pl.ds(jg * LANES, LANES)
                plsc.addupdate(o[_b4].at[r, sl], g[_b2][r, sl])
            return carry

        lax.fori_loop(0, CHUNK, addbody, 0)

        if i + 2 < NCHUNK:
            hg[i + 2] = pltpu.async_copy(
                table_hbm.at[idx_v.at[_chunk_of(i + 2)]], g[b2], gsem[b2])
        ho[i] = pltpu.async_copy(
            o[b4], out_hbm.at[pl.ds(base + _chunk_of(i) * CHUNK, CHUNK)],
            osem[b4])
        if 2 <= i and i + 2 < NCHUNK:
            ho[i - 2].wait()
            hpf[i + 2] = pltpu.async_copy(
                pe_sh.at[pl.ds(_pe_off(i + 2), CHUNK)],
                o[(i + 2) % 4], osem[(i + 2) % 4])

    ho[NCHUNK - 4].wait()
    ho[NCHUNK - 3].wait()
    ho[NCHUNK - 2].wait()
    ho[NCHUNK - 1].wait()


def kernel(x, table):
    pe = _positional_encoding()
    xf = x.astype(jnp.int32).reshape(NW, NCHUNK, CHUNK)
    out = _emb_kernel(xf, table, pe)
    return out.reshape(BATCH, SEQ, D_MODEL)


# speedup vs baseline: 1.6062x; 1.0834x over previous
import functools

import numpy as np

import jax
import jax.numpy as jnp
from jax import lax
from jax.experimental import pallas as pl
from jax.experimental.pallas import tpu as pltpu
from jax.experimental.pallas import tpu_sc as plsc

VOCAB = 32000
D_MODEL = 512
BATCH = 64
SEQ = 512

NC = 2
NS = 16
NW = NC * NS
ROWS = BATCH * SEQ
RPW = ROWS // NW
CHUNK = 32
NCHUNK = RPW // CHUNK
PCHUNK = SEQ // CHUNK
LANES = 16
GRP = D_MODEL // LANES
NGROUPS = NCHUNK // 4


def _positional_encoding():
    pos = np.arange(SEQ, dtype=np.float32)[:, None]
    i = np.arange(D_MODEL, dtype=np.float32)[None, :]
    angle = (pos / np.power(np.float32(10000.0),
                            np.float32(2.0) * i / np.float32(D_MODEL),
                            dtype=np.float32)).astype(np.float32)
    even = (np.arange(D_MODEL) % 2 == 0)[None, :]
    return np.where(even, np.sin(angle), np.cos(angle)).astype(np.float32)


_mesh = plsc.VectorSubcoreMesh(core_axis_name="c", subcore_axis_name="s")


@functools.partial(
    pl.kernel,
    mesh=_mesh,
    out_type=jax.ShapeDtypeStruct((ROWS, D_MODEL), jnp.float32),
    scratch_types=[
        pltpu.VMEM((NCHUNK, CHUNK), jnp.int32),
        pltpu.VMEM((CHUNK, D_MODEL), jnp.float32),
        pltpu.VMEM((CHUNK, D_MODEL), jnp.float32),
        pltpu.VMEM((CHUNK, D_MODEL), jnp.float32),
        pltpu.VMEM((CHUNK, D_MODEL), jnp.float32),
        pltpu.VMEM((CHUNK, D_MODEL), jnp.float32),
        pltpu.VMEM((CHUNK, D_MODEL), jnp.float32),
        pltpu.VMEM_SHARED((SEQ, D_MODEL), jnp.float32),
        pltpu.SemaphoreType.DMA,
        pltpu.SemaphoreType.DMA,
        pltpu.SemaphoreType.DMA,
        pltpu.SemaphoreType.DMA,
        pltpu.SemaphoreType.DMA,
        pltpu.SemaphoreType.DMA,
    ],
)
def _emb_kernel(x_hbm, table_hbm, pe_hbm, out_hbm, idx_v,
                g0, g1, o0, o1, o2, o3, pe_sh,
                gs0, gs1, os0, os1, os2, os3):
    wid = lax.axis_index("s") * NC + lax.axis_index("c")
    sid = lax.axis_index("s")
    base = wid * RPW
    g = (g0, g1)
    o = (o0, o1, o2, o3)
    gsem = (gs0, gs1)
    osem = (os0, os1, os2, os3)

    def chunk_of(i):
        return (i % 2) * PCHUNK + i // 2

    def start_gather(i, b2):
        pltpu.make_async_copy(
            table_hbm.at[idx_v.at[chunk_of(i)]], g[b2], gsem[b2]).start()

    def wait_gather(i, b2):
        pltpu.make_async_copy(
            table_hbm.at[idx_v.at[chunk_of(i)]], g[b2], gsem[b2]).wait()

    def start_prefill(i, b4):
        pltpu.make_async_copy(
            pe_sh.at[pl.ds((i // 2) * CHUNK, CHUNK)], o[b4], osem[b4]).start()

    def wait_osem(b4):
        pltpu.make_async_copy(
            pe_sh.at[pl.ds(0, CHUNK)], o[b4], osem[b4]).wait()

    def start_write(i, b4):
        pltpu.make_async_copy(
            o[b4], out_hbm.at[pl.ds(base + chunk_of(i) * CHUNK, CHUNK)],
            osem[b4]).start()

    def add_pass(b2, b4):
        def addbody(r, carry):
            for jg in range(GRP):
                sl = pl.ds(jg * LANES, LANES)
                plsc.addupdate(o[b4].at[r, sl], g[b2][r, sl])
            return carry
        lax.fori_loop(0, CHUNK, addbody, 0)

    pltpu.sync_copy(x_hbm.at[wid], idx_v)

    @pl.when(sid == 0)
    def _():
        pltpu.sync_copy(pe_hbm, pe_sh)

    plsc.subcore_barrier()

    for i in range(4):
        start_prefill(i, i % 4)
    for i in range(2):
        start_gather(i, i % 2)
    for i in range(4):
        wait_gather(i, i % 2)
        wait_osem(i % 4)
        add_pass(i % 2, i % 4)
        start_gather(i + 2, i % 2)
        start_write(i, i % 4)
        if i >= 2:
            wait_osem((i + 2) % 4)
            start_prefill(i + 2, (i + 2) % 4)

    def group(gi, carry):
        i0 = 4 * gi
        for j in range(4):
            i = i0 + j
            b2 = j % 2
            b4 = j % 4
            wait_gather(i, b2)
            wait_osem(b4)
            add_pass(b2, b4)
            start_gather(i + 2, b2)
            start_write(i, b4)
            wait_osem((j + 2) % 4)
            start_prefill(i + 2, (j + 2) % 4)
        return carry

    lax.fori_loop(1, NGROUPS - 1, group, 0)

    for i in range(NCHUNK - 4, NCHUNK):
        j = i % 4
        wait_gather(i, j % 2)
        wait_osem(j % 4)
        add_pass(j % 2, j % 4)
        if i + 2 < NCHUNK:
            start_gather(i + 2, j % 2)
        start_write(i, j % 4)
        if i + 2 < NCHUNK:
            wait_osem((j + 2) % 4)
            start_prefill(i + 2, (j + 2) % 4)

    for i in range(NCHUNK - 4, NCHUNK):
        wait_osem(i % 4)


def kernel(x, table):
    pe = _positional_encoding()
    xf = x.astype(jnp.int32).reshape(NW, NCHUNK, CHUNK)
    out = _emb_kernel(xf, table, pe)
    return out.reshape(BATCH, SEQ, D_MODEL)


# speedup vs baseline: 1.6869x; 1.0503x over previous
import functools

import numpy as np

import jax
import jax.numpy as jnp
from jax import lax
from jax.experimental import pallas as pl
from jax.experimental.pallas import tpu as pltpu
from jax.experimental.pallas import tpu_sc as plsc

VOCAB = 32000
D_MODEL = 512
BATCH = 64
SEQ = 512

NC = 2
NS = 16
NW = NC * NS
ROWS = BATCH * SEQ
RPW = ROWS // NW
CHUNK = 32
NCHUNK = RPW // CHUNK
PCHUNK = SEQ // CHUNK
LANES = 16
GRP = D_MODEL // LANES
NGROUPS = NCHUNK // 4


def _positional_encoding():
    pos = np.arange(SEQ, dtype=np.float32)[:, None]
    i = np.arange(D_MODEL, dtype=np.float32)[None, :]
    angle = (pos / np.power(np.float32(10000.0),
                            np.float32(2.0) * i / np.float32(D_MODEL),
                            dtype=np.float32)).astype(np.float32)
    even = (np.arange(D_MODEL) % 2 == 0)[None, :]
    return np.where(even, np.sin(angle), np.cos(angle)).astype(np.float32)


_mesh = plsc.VectorSubcoreMesh(core_axis_name="c", subcore_axis_name="s")


@functools.partial(
    pl.kernel,
    mesh=_mesh,
    out_type=jax.ShapeDtypeStruct((ROWS, D_MODEL), jnp.float32),
    scratch_types=[
        pltpu.VMEM((NCHUNK, CHUNK), jnp.int32),
        pltpu.VMEM((CHUNK, D_MODEL), jnp.float32),
        pltpu.VMEM((CHUNK, D_MODEL), jnp.float32),
        pltpu.VMEM((CHUNK, D_MODEL), jnp.float32),
        pltpu.VMEM((CHUNK, D_MODEL), jnp.float32),
        pltpu.VMEM((CHUNK, D_MODEL), jnp.float32),
        pltpu.VMEM((CHUNK, D_MODEL), jnp.float32),
        pltpu.VMEM_SHARED((SEQ, D_MODEL), jnp.float32),
        pltpu.SemaphoreType.DMA,
        pltpu.SemaphoreType.DMA,
        pltpu.SemaphoreType.DMA,
        pltpu.SemaphoreType.DMA,
        pltpu.SemaphoreType.DMA,
        pltpu.SemaphoreType.DMA,
    ],
)
def _emb_kernel(x_hbm, table_hbm, pe_hbm, out_hbm, idx_v,
                g0, g1, o0, o1, o2, o3, pe_sh,
                gs0, gs1, os0, os1, os2, os3):
    wid = lax.axis_index("s") * NC + lax.axis_index("c")
    sid = lax.axis_index("s")
    base = wid * RPW
    g = (g0, g1)
    o = (o0, o1, o2, o3)
    gsem = (gs0, gs1)
    osem = (os0, os1, os2, os3)

    def chunk_of(i):
        return (i % 2) * PCHUNK + i // 2

    def start_gather(i, b2):
        pltpu.make_async_copy(
            table_hbm.at[idx_v.at[chunk_of(i)]], g[b2], gsem[b2]).start()

    def wait_gather(i, b2):
        pltpu.make_async_copy(
            table_hbm.at[idx_v.at[chunk_of(i)]], g[b2], gsem[b2]).wait()

    def start_prefill(i, b4):
        pltpu.make_async_copy(
            pe_sh.at[pl.ds((i // 2) * CHUNK, CHUNK)], o[b4], osem[b4]).start()

    def wait_osem(b4):
        pltpu.make_async_copy(
            pe_sh.at[pl.ds(0, CHUNK)], o[b4], osem[b4]).wait()

    def start_write(i, b4):
        pltpu.make_async_copy(
            o[b4], out_hbm.at[pl.ds(base + chunk_of(i) * CHUNK, CHUNK)],
            osem[b4]).start()

    def add_pass(b2, b4):
        def addbody(r, carry):
            for jg in range(GRP):
                sl = pl.ds(jg * LANES, LANES)
                plsc.addupdate(o[b4].at[r, sl], g[b2][r, sl])
            return carry
        lax.fori_loop(0, CHUNK, addbody, 0)

    pltpu.sync_copy(x_hbm.at[wid], idx_v)
    for i in range(2):
        start_gather(i, i % 2)

    pltpu.sync_copy(pe_hbm.at[pl.ds(sid * CHUNK, CHUNK)],
                    pe_sh.at[pl.ds(sid * CHUNK, CHUNK)])
    plsc.subcore_barrier()

    for i in range(4):
        start_prefill(i, i % 4)

    def group(gi, carry):
        i0 = 4 * gi
        for j in range(4):
            i = i0 + j
            b2 = j % 2
            b4 = j % 4
            wait_gather(i, b2)
            wait_osem(b4)
            add_pass(b2, b4)

            @pl.when(i + 2 < NCHUNK)
            def _():
                start_gather(i + 2, b2)

            start_write(i, b4)

            @pl.when(jnp.logical_and(2 <= i, i + 2 < NCHUNK))
            def _():
                wait_osem((j + 2) % 4)
                start_prefill(i + 2, (j + 2) % 4)
        return carry

    lax.fori_loop(0, NGROUPS, group, 0)

    for i in range(NCHUNK - 4, NCHUNK):
        wait_osem(i % 4)


def kernel(x, table):
    pe = _positional_encoding()
    xf = x.astype(jnp.int32).reshape(NW, NCHUNK, CHUNK)
    out = _emb_kernel(xf, table, pe)
    return out.reshape(BATCH, SEQ, D_MODEL)
